# Initial kernel scaffold; baseline (speedup 1.0000x reference)
import jax
import jax.numpy as jnp
from jax.experimental import pallas as pl


def kernel(P_vec, adj):
    raise NotImplementedError("write your pallas kernel here")


# speedup vs baseline: 17.4126x; 17.4126x over previous
import jax
import jax.numpy as jnp
from jax import lax
from jax.experimental import pallas as pl
from jax.experimental.pallas import tpu as pltpu


_ROWS_PER_STEP = 8
_TILE = 256


def _phase1_body(n, p_hbm, l_hbm, sem):
    g = pl.program_id(0)
    base = g * _ROWS_PER_STEP

    def copy(k):
        r = base + k
        off = pl.multiple_of((r * (r + 1)) // 2, 128)
        return pltpu.make_async_copy(
            p_hbm.at[pl.ds(off, n)], l_hbm.at[pl.ds(r * n, n)], sem)

    for k in range(_ROWS_PER_STEP):
        copy(k).start()
    for k in range(_ROWS_PER_STEP):
        copy(k).wait()


def _phase2_body(t, l_ref, a_ref, o_ref):
    i = pl.program_id(0)
    j = pl.program_id(1)
    l = l_ref[...]
    rows = lax.broadcasted_iota(jnp.int32, (t, t), 0) + i * t
    cols = lax.broadcasted_iota(jnp.int32, (t, t), 1) + j * t
    sym = jnp.where(cols <= rows, l, l.T)
    o_ref[...] = a_ref[...] / (1.0 + jnp.exp(-sym))


def kernel(P_vec, adj):
    n = adj.shape[0]
    t = min(_TILE, n)

    unragged = pl.pallas_call(
        lambda p, l, sem: _phase1_body(n, p, l, sem),
        grid=(n // _ROWS_PER_STEP,),
        in_specs=[pl.BlockSpec(memory_space=pl.ANY)],
        out_specs=pl.BlockSpec(memory_space=pl.ANY),
        out_shape=jax.ShapeDtypeStruct((n * n,), jnp.float32),
        scratch_shapes=[pltpu.SemaphoreType.DMA],
    )
    L = unragged(P_vec).reshape(n, n)

    symm = pl.pallas_call(
        lambda l, a, o: _phase2_body(t, l, a, o),
        grid=(n // t, n // t),
        in_specs=[
            pl.BlockSpec((t, t), lambda i, j: (jnp.maximum(i, j), jnp.minimum(i, j))),
            pl.BlockSpec((t, t), lambda i, j: (i, j)),
        ],
        out_specs=pl.BlockSpec((t, t), lambda i, j: (i, j)),
        out_shape=jax.ShapeDtypeStruct((n, n), jnp.float32),
    )
    return symm(L, adj)


# speedup vs baseline: 143.2470x; 8.2266x over previous
import functools

import jax
import jax.numpy as jnp
from jax import lax
from jax.experimental import pallas as pl
from jax.experimental.pallas import tpu as pltpu
from jax.experimental.pallas import tpu_sc as plsc


_TILE = 256
_RING = 4


def _phase1_sc_body(n, n_workers, p_hbm, l_hbm, buf, in_sem, out_sem):
    wid = lax.axis_index("s") * 2 + lax.axis_index("c")
    rows_per = n // n_workers
    base = wid * rows_per

    def in_copy(k):
        r = base + k
        off = pl.multiple_of((r * (r + 1)) // 2, 128)
        return pltpu.make_async_copy(
            p_hbm.at[pl.ds(off, n)], buf.at[k % _RING], in_sem)

    def out_copy(k):
        r = base + k
        return pltpu.make_async_copy(
            buf.at[k % _RING], l_hbm.at[pl.ds(r * n, n)], out_sem)

    in_copy(0).start()

    def body(k, carry):
        @pl.when(k + 1 < rows_per)
        def _():
            @pl.when(k + 1 >= _RING)
            def _():
                out_copy(k + 1 - _RING).wait()

            in_copy(k + 1).start()

        in_copy(k).wait()
        out_copy(k).start()
        return carry

    lax.fori_loop(0, rows_per, body, 0)
    for _ in range(min(_RING, rows_per)):
        out_copy(0).wait()


def _phase2_body(t, l_ref, a_ref, o_ref):
    i = pl.program_id(0)
    j = pl.program_id(1)
    l = l_ref[...]
    rows = lax.broadcasted_iota(jnp.int32, (t, t), 0) + i * t
    cols = lax.broadcasted_iota(jnp.int32, (t, t), 1) + j * t
    sym = jnp.where(cols <= rows, l, l.T)
    o_ref[...] = a_ref[...] / (1.0 + jnp.exp(-sym))


def kernel(P_vec, adj):
    n = adj.shape[0]
    t = min(_TILE, n)

    mesh = plsc.VectorSubcoreMesh(core_axis_name="c", subcore_axis_name="s")
    info = plsc.get_sparse_core_info()
    n_workers = info.num_cores * info.num_subcores

    unragged = functools.partial(
        pl.kernel,
        mesh=mesh,
        out_type=jax.ShapeDtypeStruct((n * n,), jnp.float32),
        scratch_types=[
            pltpu.VMEM((_RING, n), jnp.float32),
            pltpu.SemaphoreType.DMA,
            pltpu.SemaphoreType.DMA,
        ],
    )(functools.partial(_phase1_sc_body, n, n_workers))
    L = unragged(P_vec).reshape(n, n)

    symm = pl.pallas_call(
        lambda l, a, o: _phase2_body(t, l, a, o),
        grid=(n // t, n // t),
        in_specs=[
            pl.BlockSpec((t, t), lambda i, j: (jnp.maximum(i, j), jnp.minimum(i, j))),
            pl.BlockSpec((t, t), lambda i, j: (i, j)),
        ],
        out_specs=pl.BlockSpec((t, t), lambda i, j: (i, j)),
        out_shape=jax.ShapeDtypeStruct((n, n), jnp.float32),
    )
    return symm(L, adj)
